# baseline (speedup 1.0000x reference)
import functools

import jax
import jax.numpy as jnp
from jax import lax
from jax.experimental import pallas as pl
from jax.experimental.pallas import tpu as pltpu
from jax.experimental.pallas import tpu_sc as plsc

N = 10000
H = 128
NP = 10240
E = 320000
LANES = 128
BPT = 80
NW = 32
EPAD = NW * BPT * LANES
NBLK = NW * BPT
NTILES = 16
STRIPE = NP // NTILES
RB = 512
GRID = NP // RB
_mesh = plsc.VectorSubcoreMesh(core_axis_name="c", subcore_axis_name="s")
_f32 = jnp.float32


NBUF = 2
NCHUNK = 2
CBPT = BPT // NCHUNK
NG = CBPT // NBUF


@functools.partial(
    pl.kernel,
    mesh=_mesh,
    out_type=jax.ShapeDtypeStruct((2, NP, H), _f32),
    scratch_types=[
        pltpu.VMEM((CBPT, LANES), jnp.int32),
        pltpu.VMEM((CBPT, LANES), jnp.int32),
        pltpu.VMEM((LANES, H), _f32),
        pltpu.VMEM((LANES, H), _f32),
        pltpu.VMEM_SHARED((NP, H), _f32),
        pltpu.SemaphoreType.DMA,
        pltpu.SemaphoreType.DMA,
    ],
)
def _sc_propagate(y_hbm, src_hbm, dst_hbm, zeros_hbm, out_hbm,
                  src_v, dst_v, b0, b1, acc, s0, s1):
    bufs = (b0, b1)
    sems = (s0, s1)
    c = lax.axis_index("c")
    s = lax.axis_index("s")
    tid = c * NTILES + s
    pltpu.sync_copy(zeros_hbm, acc.at[pl.ds(s * STRIPE, STRIPE)])
    plsc.subcore_barrier()

    for chunk in range(NCHUNK):
        base = tid * BPT + chunk * CBPT
        pltpu.sync_copy(src_hbm.at[pl.ds(base, CBPT)], src_v)
        pltpu.sync_copy(dst_hbm.at[pl.ds(base, CBPT)], dst_v)

        def body(jj, carry):
            for b in range(NBUF):
                pltpu.async_copy(y_hbm.at[src_v.at[jj * NBUF + b]],
                                 bufs[b], sems[b])
            for b in range(NBUF):
                pltpu.make_async_copy(y_hbm.at[pl.ds(0, LANES)],
                                      bufs[b], sems[b]).wait()
                pltpu.sync_copy(bufs[b], acc.at[dst_v.at[jj * NBUF + b]],
                                add=True)
            return carry

        lax.fori_loop(0, NG, body, 0)

    plsc.subcore_barrier()
    pltpu.sync_copy(acc.at[pl.ds(s * STRIPE, STRIPE)],
                    out_hbm.at[c].at[pl.ds(s * STRIPE, STRIPE)])


def _dot(a, b):
    return jnp.dot(a, b, preferred_element_type=_f32,
                   precision=lax.Precision.HIGHEST)


def _tc_first_body(deg_ref, x_ref, w_ref, y_ref, dinv_ref):
    i = pl.program_id(0)
    deg = deg_ref[0, :, 0:1] + deg_ref[1, :, 0:1] + 1.0
    rows = i * RB + lax.broadcasted_iota(jnp.int32, (RB, 1), 0)
    dinv = jnp.where(rows < N, lax.rsqrt(deg), 0.0)
    y_ref[:] = dinv * _dot(x_ref[:], w_ref[:])
    dinv_ref[:] = dinv


_tc_first = pl.pallas_call(
    _tc_first_body,
    grid=(GRID,),
    in_specs=[pl.BlockSpec((2, RB, H), lambda i: (0, i, 0)),
              pl.BlockSpec((RB, H), lambda i: (i, 0)),
              pl.BlockSpec((H, H), lambda i: (0, 0))],
    out_specs=[pl.BlockSpec((RB, H), lambda i: (i, 0)),
               pl.BlockSpec((RB, 1), lambda i: (i, 0))],
    out_shape=[jax.ShapeDtypeStruct((NP, H), _f32),
               jax.ShapeDtypeStruct((NP, 1), _f32)],
)


def _tc_mid_body(p_ref, y_ref, dinv_ref, b_ref, w_ref, h_ref, y2_ref):
    d = dinv_ref[:]
    h = jnp.maximum(d * (p_ref[0] + p_ref[1] + y_ref[:]) + b_ref[:], 0.0)
    h_ref[:] = h
    y2_ref[:] = d * _dot(h, w_ref[:])


_tc_mid = pl.pallas_call(
    _tc_mid_body,
    grid=(GRID,),
    in_specs=[pl.BlockSpec((2, RB, H), lambda i: (0, i, 0)),
              pl.BlockSpec((RB, H), lambda i: (i, 0)),
              pl.BlockSpec((RB, 1), lambda i: (i, 0)),
              pl.BlockSpec((1, H), lambda i: (0, 0)),
              pl.BlockSpec((H, H), lambda i: (0, 0))],
    out_specs=[pl.BlockSpec((RB, H), lambda i: (i, 0)),
               pl.BlockSpec((RB, H), lambda i: (i, 0))],
    out_shape=[jax.ShapeDtypeStruct((NP, H), _f32),
               jax.ShapeDtypeStruct((NP, H), _f32)],
)


def _tc_last_body(p_ref, y_ref, dinv_ref, b_ref, h_ref):
    d = dinv_ref[:]
    h_ref[:] = jnp.maximum(
        d * (p_ref[0] + p_ref[1] + y_ref[:]) + b_ref[:], 0.0)


_tc_last = pl.pallas_call(
    _tc_last_body,
    grid=(GRID,),
    in_specs=[pl.BlockSpec((2, RB, H), lambda i: (0, i, 0)),
              pl.BlockSpec((RB, H), lambda i: (i, 0)),
              pl.BlockSpec((RB, 1), lambda i: (i, 0)),
              pl.BlockSpec((1, H), lambda i: (0, 0))],
    out_specs=pl.BlockSpec((RB, H), lambda i: (i, 0)),
    out_shape=jax.ShapeDtypeStruct((NP, H), _f32),
)


def _tc_final_body(h0_ref, h1_ref, h2_ref, h3_ref, wl_ref, bl_ref, o_ref):
    z = (bl_ref[:]
         + _dot(h0_ref[:], wl_ref[0:H])
         + _dot(h1_ref[:], wl_ref[H:2 * H])
         + _dot(h2_ref[:], wl_ref[2 * H:3 * H])
         + _dot(h3_ref[:], wl_ref[3 * H:4 * H]))
    z = z - jnp.max(z, axis=-1, keepdims=True)
    o_ref[:] = z - jnp.log(jnp.sum(jnp.exp(z), axis=-1, keepdims=True))


_tc_final = pl.pallas_call(
    _tc_final_body,
    grid=(GRID,),
    in_specs=[pl.BlockSpec((RB, H), lambda i: (i, 0)),
              pl.BlockSpec((RB, H), lambda i: (i, 0)),
              pl.BlockSpec((RB, H), lambda i: (i, 0)),
              pl.BlockSpec((RB, H), lambda i: (i, 0)),
              pl.BlockSpec((4 * H, H), lambda i: (0, 0)),
              pl.BlockSpec((1, H), lambda i: (0, 0))],
    out_specs=pl.BlockSpec((RB, H), lambda i: (i, 0)),
    out_shape=jax.ShapeDtypeStruct((NP, H), _f32),
)


def kernel(x, edge_index, W0, b0, W1, b1, W2, b2, W3, b3, Wl, bl):
    x_pad = jnp.zeros((NP, H), _f32).at[:N, :].set(x)
    fill = jnp.full((EPAD - E,), N, jnp.int32)
    src2 = jnp.concatenate([edge_index[0], fill]).reshape(NBLK, LANES)
    dst2 = jnp.concatenate([edge_index[1], fill]).reshape(NBLK, LANES)
    zeros_p = jnp.zeros((STRIPE, H), _f32)
    ones_y = jnp.ones((NP, H), _f32)

    degp = _sc_propagate(ones_y, src2, dst2, zeros_p)
    y, dinv = _tc_first(degp, x_pad, W0)

    hs = []
    for (b_cur, w_next) in ((b0, W1), (b1, W2), (b2, W3)):
        p = _sc_propagate(y, src2, dst2, zeros_p)
        h, y = _tc_mid(p, y, dinv, b_cur.reshape(1, H), w_next)
        hs.append(h)
    p = _sc_propagate(y, src2, dst2, zeros_p)
    hs.append(_tc_last(p, y, dinv, b3.reshape(1, H)))

    out = _tc_final(hs[0], hs[1], hs[2], hs[3], Wl, bl.reshape(1, H))
    return out[:N]


# speedup vs baseline: 1.2141x; 1.2141x over previous
import functools

import jax
import jax.numpy as jnp
from jax import lax
from jax.experimental import pallas as pl
from jax.experimental.pallas import tpu as pltpu
from jax.experimental.pallas import tpu_sc as plsc

N = 10000
H = 128
NP = 10240
E = 320000
LANES = 128
BPT = 80
NW = 32
EPAD = NW * BPT * LANES
NBLK = NW * BPT
NTILES = 16
STRIPE = NP // NTILES
RB = 512
GRID = NP // RB
_mesh = plsc.VectorSubcoreMesh(core_axis_name="c", subcore_axis_name="s")
_f32 = jnp.float32


@functools.partial(
    pl.kernel,
    mesh=_mesh,
    out_type=jax.ShapeDtypeStruct((2, NP, H), _f32),
    scratch_types=[
        pltpu.VMEM((BPT, LANES), jnp.int32),
        pltpu.VMEM((LANES, H), _f32),
        pltpu.VMEM_SHARED((NP, H), _f32),
    ],
)
def _sc_degree(dst_hbm, ones_hbm, zeros_hbm, out_hbm, dst_v, ones_v, acc):
    c = lax.axis_index("c")
    s = lax.axis_index("s")
    tid = c * NTILES + s
    pltpu.sync_copy(zeros_hbm, acc.at[pl.ds(s * STRIPE, STRIPE)])
    plsc.subcore_barrier()
    pltpu.sync_copy(dst_hbm.at[pl.ds(tid * BPT, BPT)], dst_v)
    pltpu.sync_copy(ones_hbm, ones_v)

    def body(j, carry):
        pltpu.sync_copy(ones_v, acc.at[dst_v.at[j]], add=True)
        return carry

    lax.fori_loop(0, BPT, body, 0)
    plsc.subcore_barrier()
    pltpu.sync_copy(acc.at[pl.ds(s * STRIPE, STRIPE)],
                    out_hbm.at[c].at[pl.ds(s * STRIPE, STRIPE)])

NBUF = 2
NCHUNK = 2
CBPT = BPT // NCHUNK
NG = CBPT // NBUF


@functools.partial(
    pl.kernel,
    mesh=_mesh,
    out_type=jax.ShapeDtypeStruct((2, NP, H), _f32),
    scratch_types=[
        pltpu.VMEM((CBPT, LANES), jnp.int32),
        pltpu.VMEM((CBPT, LANES), jnp.int32),
        pltpu.VMEM((LANES, H), _f32),
        pltpu.VMEM((LANES, H), _f32),
        pltpu.VMEM_SHARED((NP, H), _f32),
        pltpu.SemaphoreType.DMA,
        pltpu.SemaphoreType.DMA,
    ],
)
def _sc_propagate(y_hbm, src_hbm, dst_hbm, zeros_hbm, out_hbm,
                  src_v, dst_v, b0, b1, acc, s0, s1):
    bufs = (b0, b1)
    sems = (s0, s1)
    c = lax.axis_index("c")
    s = lax.axis_index("s")
    tid = c * NTILES + s
    pltpu.sync_copy(zeros_hbm, acc.at[pl.ds(s * STRIPE, STRIPE)])
    plsc.subcore_barrier()

    for chunk in range(NCHUNK):
        base = tid * BPT + chunk * CBPT
        pltpu.sync_copy(src_hbm.at[pl.ds(base, CBPT)], src_v)
        pltpu.sync_copy(dst_hbm.at[pl.ds(base, CBPT)], dst_v)

        def body(jj, carry):
            for b in range(NBUF):
                pltpu.async_copy(y_hbm.at[src_v.at[jj * NBUF + b]],
                                 bufs[b], sems[b])
            for b in range(NBUF):
                pltpu.make_async_copy(y_hbm.at[pl.ds(0, LANES)],
                                      bufs[b], sems[b]).wait()
                pltpu.sync_copy(bufs[b], acc.at[dst_v.at[jj * NBUF + b]],
                                add=True)
            return carry

        lax.fori_loop(0, NG, body, 0)

    plsc.subcore_barrier()
    pltpu.sync_copy(acc.at[pl.ds(s * STRIPE, STRIPE)],
                    out_hbm.at[c].at[pl.ds(s * STRIPE, STRIPE)])


def _dot(a, b):
    return jnp.dot(a, b, preferred_element_type=_f32,
                   precision=lax.Precision.HIGHEST)


def _tc_first_body(deg_ref, x_ref, w_ref, y_ref, dinv_ref):
    i = pl.program_id(0)
    deg = deg_ref[0, :, 0:1] + deg_ref[1, :, 0:1] + 1.0
    rows = i * RB + lax.broadcasted_iota(jnp.int32, (RB, 1), 0)
    dinv = jnp.where(rows < N, lax.rsqrt(deg), 0.0)
    y_ref[:] = dinv * _dot(x_ref[:], w_ref[:])
    dinv_ref[:] = dinv


_tc_first = pl.pallas_call(
    _tc_first_body,
    grid=(GRID,),
    in_specs=[pl.BlockSpec((2, RB, H), lambda i: (0, i, 0)),
              pl.BlockSpec((RB, H), lambda i: (i, 0)),
              pl.BlockSpec((H, H), lambda i: (0, 0))],
    out_specs=[pl.BlockSpec((RB, H), lambda i: (i, 0)),
               pl.BlockSpec((RB, 1), lambda i: (i, 0))],
    out_shape=[jax.ShapeDtypeStruct((NP, H), _f32),
               jax.ShapeDtypeStruct((NP, 1), _f32)],
)


def _tc_mid_body(p_ref, y_ref, dinv_ref, b_ref, w_ref, h_ref, y2_ref):
    d = dinv_ref[:]
    h = jnp.maximum(d * (p_ref[0] + p_ref[1] + y_ref[:]) + b_ref[:], 0.0)
    h_ref[:] = h
    y2_ref[:] = d * _dot(h, w_ref[:])


_tc_mid = pl.pallas_call(
    _tc_mid_body,
    grid=(GRID,),
    in_specs=[pl.BlockSpec((2, RB, H), lambda i: (0, i, 0)),
              pl.BlockSpec((RB, H), lambda i: (i, 0)),
              pl.BlockSpec((RB, 1), lambda i: (i, 0)),
              pl.BlockSpec((1, H), lambda i: (0, 0)),
              pl.BlockSpec((H, H), lambda i: (0, 0))],
    out_specs=[pl.BlockSpec((RB, H), lambda i: (i, 0)),
               pl.BlockSpec((RB, H), lambda i: (i, 0))],
    out_shape=[jax.ShapeDtypeStruct((NP, H), _f32),
               jax.ShapeDtypeStruct((NP, H), _f32)],
)


def _tc_last_body(p_ref, y_ref, dinv_ref, b_ref, h_ref):
    d = dinv_ref[:]
    h_ref[:] = jnp.maximum(
        d * (p_ref[0] + p_ref[1] + y_ref[:]) + b_ref[:], 0.0)


_tc_last = pl.pallas_call(
    _tc_last_body,
    grid=(GRID,),
    in_specs=[pl.BlockSpec((2, RB, H), lambda i: (0, i, 0)),
              pl.BlockSpec((RB, H), lambda i: (i, 0)),
              pl.BlockSpec((RB, 1), lambda i: (i, 0)),
              pl.BlockSpec((1, H), lambda i: (0, 0))],
    out_specs=pl.BlockSpec((RB, H), lambda i: (i, 0)),
    out_shape=jax.ShapeDtypeStruct((NP, H), _f32),
)


def _tc_final_body(h0_ref, h1_ref, h2_ref, h3_ref, wl_ref, bl_ref, o_ref):
    z = (bl_ref[:]
         + _dot(h0_ref[:], wl_ref[0:H])
         + _dot(h1_ref[:], wl_ref[H:2 * H])
         + _dot(h2_ref[:], wl_ref[2 * H:3 * H])
         + _dot(h3_ref[:], wl_ref[3 * H:4 * H]))
    z = z - jnp.max(z, axis=-1, keepdims=True)
    o_ref[:] = z - jnp.log(jnp.sum(jnp.exp(z), axis=-1, keepdims=True))


_tc_final = pl.pallas_call(
    _tc_final_body,
    grid=(GRID,),
    in_specs=[pl.BlockSpec((RB, H), lambda i: (i, 0)),
              pl.BlockSpec((RB, H), lambda i: (i, 0)),
              pl.BlockSpec((RB, H), lambda i: (i, 0)),
              pl.BlockSpec((RB, H), lambda i: (i, 0)),
              pl.BlockSpec((4 * H, H), lambda i: (0, 0)),
              pl.BlockSpec((1, H), lambda i: (0, 0))],
    out_specs=pl.BlockSpec((RB, H), lambda i: (i, 0)),
    out_shape=jax.ShapeDtypeStruct((NP, H), _f32),
)


def kernel(x, edge_index, W0, b0, W1, b1, W2, b2, W3, b3, Wl, bl):
    x_pad = jnp.zeros((NP, H), _f32).at[:N, :].set(x)
    fill = jnp.full((EPAD - E,), N, jnp.int32)
    src2 = jnp.concatenate([edge_index[0], fill]).reshape(NBLK, LANES)
    dst2 = jnp.concatenate([edge_index[1], fill]).reshape(NBLK, LANES)
    zeros_p = jnp.zeros((STRIPE, H), _f32)
    ones_d = jnp.ones((LANES, H), _f32)

    degp = _sc_degree(dst2, ones_d, zeros_p)
    y, dinv = _tc_first(degp, x_pad, W0)

    hs = []
    for (b_cur, w_next) in ((b0, W1), (b1, W2), (b2, W3)):
        p = _sc_propagate(y, src2, dst2, zeros_p)
        h, y = _tc_mid(p, y, dinv, b_cur.reshape(1, H), w_next)
        hs.append(h)
    p = _sc_propagate(y, src2, dst2, zeros_p)
    hs.append(_tc_last(p, y, dinv, b3.reshape(1, H)))

    out = _tc_final(hs[0], hs[1], hs[2], hs[3], Wl, bl.reshape(1, H))
    return out[:N]


# speedup vs baseline: 1.3991x; 1.1525x over previous
import functools

import jax
import jax.numpy as jnp
from jax import lax
from jax.experimental import pallas as pl
from jax.experimental.pallas import tpu as pltpu
from jax.experimental.pallas import tpu_sc as plsc

N = 10000
H = 128
NP = 10240
E = 320000
LANES = 128
BPT = 80
NW = 32
EPAD = NW * BPT * LANES
NBLK = NW * BPT
NTILES = 16
STRIPE = NP // NTILES
RB = 512
GRID = NP // RB
_mesh = plsc.VectorSubcoreMesh(core_axis_name="c", subcore_axis_name="s")
_f32 = jnp.float32


@functools.partial(
    pl.kernel,
    mesh=_mesh,
    out_type=jax.ShapeDtypeStruct((2, NP, H), _f32),
    scratch_types=[
        pltpu.VMEM((BPT, LANES), jnp.int32),
        pltpu.VMEM((LANES, H), _f32),
        pltpu.VMEM_SHARED((NP, H), _f32),
    ],
)
def _sc_degree(dst_hbm, ones_hbm, zeros_hbm, out_hbm, dst_v, ones_v, acc):
    c = lax.axis_index("c")
    s = lax.axis_index("s")
    tid = c * NTILES + s
    pltpu.sync_copy(zeros_hbm, acc.at[pl.ds(s * STRIPE, STRIPE)])
    plsc.subcore_barrier()
    pltpu.sync_copy(dst_hbm.at[pl.ds(tid * BPT, BPT)], dst_v)
    pltpu.sync_copy(ones_hbm, ones_v)

    def body(j, carry):
        pltpu.sync_copy(ones_v, acc.at[dst_v.at[j]], add=True)
        return carry

    lax.fori_loop(0, BPT, body, 0)
    plsc.subcore_barrier()
    pltpu.sync_copy(acc.at[pl.ds(s * STRIPE, STRIPE)],
                    out_hbm.at[c].at[pl.ds(s * STRIPE, STRIPE)])

NBUF = 2
CB = 40
NCF = 3
NCS = 1
FB = NCF * CB
SB = NCS * CB


@functools.partial(
    pl.kernel,
    mesh=_mesh,
    out_type=jax.ShapeDtypeStruct((2, NP, H), _f32),
    scratch_types=[
        pltpu.VMEM((CB, LANES), jnp.int32),
        pltpu.VMEM((CB, LANES), jnp.int32),
        pltpu.VMEM((LANES, H), _f32),
        pltpu.VMEM((LANES, H), _f32),
        pltpu.VMEM_SHARED((NP, H), _f32),
        pltpu.SemaphoreType.DMA,
        pltpu.SemaphoreType.DMA,
    ],
)
def _sc_propagate(y_hbm, src_hbm, dst_hbm, zeros_hbm, out_hbm,
                  src_v, dst_v, b0, b1, acc, s0, s1):
    bufs = (b0, b1)
    sems = (s0, s1)
    c = lax.axis_index("c")
    s = lax.axis_index("s")
    pltpu.sync_copy(zeros_hbm, acc.at[pl.ds(s * STRIPE, STRIPE)])
    plsc.subcore_barrier()
    nc = jnp.where(c == 0, NCF, NCS)
    base = jnp.where(c == 0, s * FB, 16 * FB + s * SB)

    def chunk_body(k, carry):
        off = pl.multiple_of(base + k * CB, 8)
        pltpu.sync_copy(src_hbm.at[pl.ds(off, CB)], src_v)
        pltpu.sync_copy(dst_hbm.at[pl.ds(off, CB)], dst_v)

        def body(jj, carry2):
            for b in range(NBUF):
                pltpu.async_copy(y_hbm.at[src_v.at[jj * NBUF + b]],
                                 bufs[b], sems[b])
            for b in range(NBUF):
                pltpu.make_async_copy(y_hbm.at[pl.ds(0, LANES)],
                                      bufs[b], sems[b]).wait()
                pltpu.sync_copy(bufs[b], acc.at[dst_v.at[jj * NBUF + b]],
                                add=True)
            return carry2

        lax.fori_loop(0, CB // NBUF, body, 0)
        return carry

    lax.fori_loop(0, nc, chunk_body, 0)
    plsc.subcore_barrier()
    pltpu.sync_copy(acc.at[pl.ds(s * STRIPE, STRIPE)],
                    out_hbm.at[c].at[pl.ds(s * STRIPE, STRIPE)])


def _dot(a, b):
    return jnp.dot(a, b, preferred_element_type=_f32,
                   precision=lax.Precision.HIGHEST)


def _tc_first_body(deg_ref, x_ref, w_ref, y_ref, dinv_ref):
    i = pl.program_id(0)
    deg = deg_ref[0, :, 0:1] + deg_ref[1, :, 0:1] + 1.0
    rows = i * RB + lax.broadcasted_iota(jnp.int32, (RB, 1), 0)
    dinv = jnp.where(rows < N, lax.rsqrt(deg), 0.0)
    y_ref[:] = dinv * _dot(x_ref[:], w_ref[:])
    dinv_ref[:] = dinv


_tc_first = pl.pallas_call(
    _tc_first_body,
    grid=(GRID,),
    in_specs=[pl.BlockSpec((2, RB, H), lambda i: (0, i, 0)),
              pl.BlockSpec((RB, H), lambda i: (i, 0)),
              pl.BlockSpec((H, H), lambda i: (0, 0))],
    out_specs=[pl.BlockSpec((RB, H), lambda i: (i, 0)),
               pl.BlockSpec((RB, 1), lambda i: (i, 0))],
    out_shape=[jax.ShapeDtypeStruct((NP, H), _f32),
               jax.ShapeDtypeStruct((NP, 1), _f32)],
)


def _tc_mid_body(p_ref, y_ref, dinv_ref, b_ref, w_ref, h_ref, y2_ref):
    d = dinv_ref[:]
    h = jnp.maximum(d * (p_ref[0] + p_ref[1] + y_ref[:]) + b_ref[:], 0.0)
    h_ref[:] = h
    y2_ref[:] = d * _dot(h, w_ref[:])


_tc_mid = pl.pallas_call(
    _tc_mid_body,
    grid=(GRID,),
    in_specs=[pl.BlockSpec((2, RB, H), lambda i: (0, i, 0)),
              pl.BlockSpec((RB, H), lambda i: (i, 0)),
              pl.BlockSpec((RB, 1), lambda i: (i, 0)),
              pl.BlockSpec((1, H), lambda i: (0, 0)),
              pl.BlockSpec((H, H), lambda i: (0, 0))],
    out_specs=[pl.BlockSpec((RB, H), lambda i: (i, 0)),
               pl.BlockSpec((RB, H), lambda i: (i, 0))],
    out_shape=[jax.ShapeDtypeStruct((NP, H), _f32),
               jax.ShapeDtypeStruct((NP, H), _f32)],
)


def _tc_last_body(p_ref, y_ref, dinv_ref, b_ref, h_ref):
    d = dinv_ref[:]
    h_ref[:] = jnp.maximum(
        d * (p_ref[0] + p_ref[1] + y_ref[:]) + b_ref[:], 0.0)


_tc_last = pl.pallas_call(
    _tc_last_body,
    grid=(GRID,),
    in_specs=[pl.BlockSpec((2, RB, H), lambda i: (0, i, 0)),
              pl.BlockSpec((RB, H), lambda i: (i, 0)),
              pl.BlockSpec((RB, 1), lambda i: (i, 0)),
              pl.BlockSpec((1, H), lambda i: (0, 0))],
    out_specs=pl.BlockSpec((RB, H), lambda i: (i, 0)),
    out_shape=jax.ShapeDtypeStruct((NP, H), _f32),
)


def _tc_final_body(h0_ref, h1_ref, h2_ref, h3_ref, wl_ref, bl_ref, o_ref):
    z = (bl_ref[:]
         + _dot(h0_ref[:], wl_ref[0:H])
         + _dot(h1_ref[:], wl_ref[H:2 * H])
         + _dot(h2_ref[:], wl_ref[2 * H:3 * H])
         + _dot(h3_ref[:], wl_ref[3 * H:4 * H]))
    z = z - jnp.max(z, axis=-1, keepdims=True)
    o_ref[:] = z - jnp.log(jnp.sum(jnp.exp(z), axis=-1, keepdims=True))


_tc_final = pl.pallas_call(
    _tc_final_body,
    grid=(GRID,),
    in_specs=[pl.BlockSpec((RB, H), lambda i: (i, 0)),
              pl.BlockSpec((RB, H), lambda i: (i, 0)),
              pl.BlockSpec((RB, H), lambda i: (i, 0)),
              pl.BlockSpec((RB, H), lambda i: (i, 0)),
              pl.BlockSpec((4 * H, H), lambda i: (0, 0)),
              pl.BlockSpec((1, H), lambda i: (0, 0))],
    out_specs=pl.BlockSpec((RB, H), lambda i: (i, 0)),
    out_shape=jax.ShapeDtypeStruct((NP, H), _f32),
)


def kernel(x, edge_index, W0, b0, W1, b1, W2, b2, W3, b3, Wl, bl):
    x_pad = jnp.zeros((NP, H), _f32).at[:N, :].set(x)
    fill = jnp.full((EPAD - E,), N, jnp.int32)
    src2 = jnp.concatenate([edge_index[0], fill]).reshape(NBLK, LANES)
    dst2 = jnp.concatenate([edge_index[1], fill]).reshape(NBLK, LANES)
    zeros_p = jnp.zeros((STRIPE, H), _f32)
    ones_d = jnp.ones((LANES, H), _f32)

    degp = _sc_degree(dst2, ones_d, zeros_p)
    y, dinv = _tc_first(degp, x_pad, W0)

    hs = []
    for (b_cur, w_next) in ((b0, W1), (b1, W2), (b2, W3)):
        p = _sc_propagate(y, src2, dst2, zeros_p)
        h, y = _tc_mid(p, y, dinv, b_cur.reshape(1, H), w_next)
        hs.append(h)
    p = _sc_propagate(y, src2, dst2, zeros_p)
    hs.append(_tc_last(p, y, dinv, b3.reshape(1, H)))

    out = _tc_final(hs[0], hs[1], hs[2], hs[3], Wl, bl.reshape(1, H))
    return out[:N]
